# Initial kernel scaffold; baseline (speedup 1.0000x reference)
import jax
import jax.numpy as jnp
from jax.experimental import pallas as pl


def kernel(x, gw1, gb1, gw2, gb2, wh, bh, wo, bo):
    raise NotImplementedError("write your pallas kernel here")


# speedup vs baseline: 2.3211x; 2.3211x over previous
import jax
import jax.numpy as jnp
from jax.experimental import pallas as pl
from jax.experimental.pallas import tpu as pltpu

_T, _D, _G, _H, _E = 2048, 768, 256, 1536, 8


def _gate_kernel(x_ref, gw1_ref, gb1_ref, gw2_ref, gb2_ref, w_ref, aux_ref):
    x = x_ref[...]
    gh = jnp.maximum(
        jnp.dot(x, gw1_ref[...], preferred_element_type=jnp.float32)
        + gb1_ref[...], 0.0)
    logits = (jnp.dot(gh, gw2_ref[...], preferred_element_type=jnp.float32)
              + gb2_ref[...])
    m = jnp.max(logits, axis=1, keepdims=True)
    ex = jnp.exp(logits - m)
    p = ex / jnp.sum(ex, axis=1, keepdims=True)

    lane = jax.lax.broadcasted_iota(jnp.int32, (_T, _E), 1)
    m1 = jnp.max(p, axis=1, keepdims=True)
    i1 = jnp.min(jnp.where(p == m1, lane, _E), axis=1, keepdims=True)
    pm = jnp.where(lane == i1, -1.0, p)
    m2 = jnp.max(pm, axis=1, keepdims=True)
    i2 = jnp.min(jnp.where(pm == m2, lane, _E), axis=1, keepdims=True)

    denom = m1 + m2 + 1e-9
    w = jnp.where(lane == i1, m1, 0.0) + jnp.where(lane == i2, m2, 0.0)
    w_ref[...] = w / denom

    c = jnp.where(lane == i1, 1.0, 0.0) + jnp.where(lane == i2, 1.0, 0.0)
    cnt = jnp.sum(c, axis=0, keepdims=True)
    load = cnt / (_T + 1e-9)
    lbm = jnp.mean(load)
    lbl = jnp.sum((load - lbm) ** 2) / (_E - 1)
    ent = -jnp.mean(jnp.sum(p * jnp.log(p + 1e-9), axis=1))
    aux_ref[...] = jnp.reshape(lbl + ent, (1, 1))


_BT = 1024


def _expert_kernel(x_ref, wh_ref, bh_ref, wo_ref, bo_ref, w_ref, out_ref):
    e = pl.program_id(1)
    xb = x_ref[...]
    h = jnp.maximum(
        jnp.dot(xb, wh_ref[0], preferred_element_type=jnp.float32)
        + bh_ref[0], 0.0)
    y = jnp.dot(h, wo_ref[0], preferred_element_type=jnp.float32) + bo_ref[0]
    lane = jax.lax.broadcasted_iota(jnp.int32, (_BT, _E), 1)
    wcol = jnp.sum(jnp.where(lane == e, w_ref[...], 0.0), axis=1,
                   keepdims=True)
    contrib = y * wcol

    @pl.when(e == 0)
    def _():
        out_ref[...] = contrib

    @pl.when(e > 0)
    def _():
        out_ref[...] += contrib


def kernel(x, gw1, gb1, gw2, gb2, wh, bh, wo, bo):
    gb1r = gb1.reshape(1, _G)
    gb2r = gb2.reshape(1, _E)

    w, aux = pl.pallas_call(
        _gate_kernel,
        out_shape=[
            jax.ShapeDtypeStruct((_T, _E), jnp.float32),
            jax.ShapeDtypeStruct((1, 1), jnp.float32),
        ],
    )(x, gw1, gb1r, gw2, gb2r)

    nt = _T // _BT
    out = pl.pallas_call(
        _expert_kernel,
        grid=(nt, _E),
        in_specs=[
            pl.BlockSpec((_BT, _D), lambda t, e: (t, 0)),
            pl.BlockSpec((1, _D, _H), lambda t, e: (e, 0, 0)),
            pl.BlockSpec((1, 1, _H), lambda t, e: (e, 0, 0)),
            pl.BlockSpec((1, _H, _D), lambda t, e: (e, 0, 0)),
            pl.BlockSpec((1, 1, _D), lambda t, e: (e, 0, 0)),
            pl.BlockSpec((_BT, _E), lambda t, e: (t, 0)),
        ],
        out_specs=pl.BlockSpec((_BT, _D), lambda t, e: (t, 0)),
        out_shape=jax.ShapeDtypeStruct((_T, _D), jnp.float32),
        compiler_params=pltpu.CompilerParams(
            dimension_semantics=("arbitrary", "arbitrary"),
        ),
    )(x, wh, bh.reshape(_E, 1, _H), wo, bo.reshape(_E, 1, _D), w)

    return out, aux.reshape(())
